# Initial kernel scaffold; baseline (speedup 1.0000x reference)
import jax
import jax.numpy as jnp
from jax.experimental import pallas as pl


def kernel(edge_index, u_flag, v_flag, W0a, b0a, W0b, b0b, W1a, b1a, W1b, b1b, W2a, b2a, W2b, b2b, Ws1, bs1, Ws2, bs2):
    raise NotImplementedError("write your pallas kernel here")


# speedup vs baseline: 3.9631x; 3.9631x over previous
import functools

import jax
import jax.numpy as jnp
from jax import lax
from jax.experimental import pallas as pl
from jax.experimental.pallas import tpu as pltpu
from jax.experimental.pallas import tpu_sc as plsc

_N = 50000
_NP = 50048
_E = 800000
_H = 64
_HALF = 32

_NSUB = 16
_CH = 128
_EP = 819200
_NCHT = 6400
_NCH = _NCHT // _NSUB
_GRP = 16
_NGRP = _NCH // _GRP
_NPS = _NP // _NSUB
_NP2 = 50176
_NPS2 = _NP2 // _NSUB
_TRASH = 50048


def _make_agg(F):
    mesh = plsc.VectorSubcoreMesh(core_axis_name="c", subcore_axis_name="s")

    @functools.partial(
        pl.kernel,
        mesh=mesh,
        compiler_params=pltpu.CompilerParams(use_tc_tiling_on_sc=False),
        out_type=jax.ShapeDtypeStruct((2, _NP, F), jnp.float32),
        scratch_types=[
            pltpu.VMEM((_GRP, 2, _CH), jnp.int32),
            pltpu.VMEM((_CH, F), jnp.float32),
            pltpu.VMEM_SHARED((_NP2, F), jnp.float32),
            pltpu.SemaphoreType.DMA,
            pltpu.SemaphoreType.DMA,
        ],
    )
    def k(h_st, idx_tab, zeros, out, idx_v, rows_v, agg_sh, sem, sem2):
        s = lax.axis_index("s")
        c = lax.axis_index("c")
        pltpu.sync_copy(zeros, agg_sh.at[pl.ds(s * _NPS2, _NPS2)])
        plsc.subcore_barrier()

        iota16 = lax.iota(jnp.int32, 16)
        cid0 = s * _NCH

        for cc in range(2):
            @pl.when(c == cc)
            def _():
                h_c = h_st.at[cc]

                def group(g, carry):
                    ids = iota16 + (cid0 + g * _GRP)
                    pltpu.async_copy(idx_tab.at[ids], idx_v, sem2).wait()
                    for t in range(_GRP):
                        pltpu.async_copy(
                            h_c.at[idx_v.at[t, 0]], rows_v, sem).wait()
                        pltpu.sync_copy(
                            rows_v, agg_sh.at[idx_v.at[t, 1]], add=True)
                    return carry

                lax.fori_loop(0, _NGRP, group, 0)

        plsc.subcore_barrier()
        row0 = s * _NPS
        for cc in range(2):
            @pl.when(c == cc)
            def _():
                pltpu.sync_copy(agg_sh.at[pl.ds(row0, _NPS)],
                                out.at[cc].at[pl.ds(row0, _NPS)])

    return k


_R = 2000
_G = _N // _R


def _mlp(z, Wa, ba, Wb, bb):
    z = jnp.maximum(jnp.dot(z, Wa, preferred_element_type=jnp.float32) + ba, 0.0)
    z = jnp.dot(z, Wb, preferred_element_type=jnp.float32) + bb
    return jnp.maximum(z, 0.0)


def _t0_body(feat, agg, Wa, ba, Wb, bb, out):
    z = feat[...] + agg[...]
    h = _mlp(z, Wa[...], ba[...], Wb[...], bb[...])
    out[0] = h[:, :_HALF]
    out[1] = h[:, _HALF:]


def _t1_body(h_st, agg, Wa, ba, Wb, bb, out):
    z = jnp.concatenate([h_st[0] + agg[0], h_st[1] + agg[1]], axis=1)
    h = _mlp(z, Wa[...], ba[...], Wb[...], bb[...])
    out[0] = h[:, :_HALF]
    out[1] = h[:, _HALF:]


def _t2_body(h_st, agg, Wa, ba, Wb, bb, Ws1, bs1, Ws2, bs2, score, acc):
    i = pl.program_id(0)
    z = jnp.concatenate([h_st[0] + agg[0], h_st[1] + agg[1]], axis=1)
    h = _mlp(z, Wa[...], ba[...], Wb[...], bb[...])
    blk = jnp.sum(h, axis=0, keepdims=True)

    @pl.when(i == 0)
    def _():
        acc[...] = blk

    @pl.when(i > 0)
    def _():
        acc[...] = acc[...] + blk

    @pl.when(i == _G - 1)
    def _():
        hg = acc[...] * (1.0 / _N)
        sv = jnp.maximum(
            jnp.dot(hg, Ws1[...], preferred_element_type=jnp.float32) + bs1[...],
            0.0)
        sc = jnp.dot(sv, Ws2[...], preferred_element_type=jnp.float32) + bs2[...]
        score[...] = jax.nn.sigmoid(sc)


def _full_spec(shape):
    return pl.BlockSpec(shape, lambda i: tuple(0 for _ in shape))


_t0 = pl.pallas_call(
    _t0_body,
    grid=(_G,),
    in_specs=[
        pl.BlockSpec((_R, 2), lambda i: (i, 0)),
        pl.BlockSpec((_R, 2), lambda i: (i, 0)),
        _full_spec((2, _H)),
        _full_spec((1, _H)),
        _full_spec((_H, _H)),
        _full_spec((1, _H)),
    ],
    out_specs=pl.BlockSpec((2, _R, _HALF), lambda i: (0, i, 0)),
    out_shape=jax.ShapeDtypeStruct((2, _NP, _HALF), jnp.float32),
)

_t1 = pl.pallas_call(
    _t1_body,
    grid=(_G,),
    in_specs=[
        pl.BlockSpec((2, _R, _HALF), lambda i: (0, i, 0)),
        pl.BlockSpec((2, _R, _HALF), lambda i: (0, i, 0)),
        _full_spec((_H, _H)),
        _full_spec((1, _H)),
        _full_spec((_H, _H)),
        _full_spec((1, _H)),
    ],
    out_specs=pl.BlockSpec((2, _R, _HALF), lambda i: (0, i, 0)),
    out_shape=jax.ShapeDtypeStruct((2, _NP, _HALF), jnp.float32),
)

_t2 = pl.pallas_call(
    _t2_body,
    grid=(_G,),
    in_specs=[
        pl.BlockSpec((2, _R, _HALF), lambda i: (0, i, 0)),
        pl.BlockSpec((2, _R, _HALF), lambda i: (0, i, 0)),
        _full_spec((_H, _H)),
        _full_spec((1, _H)),
        _full_spec((_H, _H)),
        _full_spec((1, _H)),
        _full_spec((_H, _H)),
        _full_spec((1, _H)),
        _full_spec((_H, 1)),
        _full_spec((1, 1)),
    ],
    out_specs=pl.BlockSpec((1, 1), lambda i: (0, 0)),
    out_shape=jax.ShapeDtypeStruct((1, 1), jnp.float32),
    scratch_shapes=[pltpu.VMEM((1, _H), jnp.float32)],
)

_F0 = 8
_agg_half = _make_agg(_HALF)
_agg_f0 = _make_agg(_F0)


def kernel(edge_index, u_flag, v_flag, W0a, b0a, W0b, b0b,
           W1a, b1a, W1b, b1b, W2a, b2a, W2b, b2b, Ws1, bs1, Ws2, bs2):
    u32 = u_flag.astype(jnp.float32)
    v32 = v_flag.astype(jnp.float32)
    flags2 = jnp.stack([u32, v32], axis=1)
    flags_st = jnp.zeros((2, _NP, _F0), jnp.float32).at[:, :_N, 0].set(
        jnp.stack([u32, v32], axis=0))
    pad = _EP - _E
    src_p = jnp.concatenate([edge_index[0], jnp.zeros((pad,), jnp.int32)])
    dst_p = jnp.concatenate(
        [edge_index[1], jnp.full((pad,), _TRASH, jnp.int32)])
    idx_tab = jnp.stack(
        [src_p.reshape(_NCHT, _CH), dst_p.reshape(_NCHT, _CH)], axis=1)
    zeros8 = jnp.zeros((_NPS2, _F0), jnp.float32)
    zeros32 = jnp.zeros((_NPS2, _HALF), jnp.float32)

    agg0 = _agg_f0(flags_st, idx_tab, zeros8)
    agg0_2 = agg0[:, :_N, 0].T
    h1 = _t0(flags2, agg0_2, W0a, b0a.reshape(1, _H), W0b, b0b.reshape(1, _H))

    agg1 = _agg_half(h1, idx_tab, zeros32)
    h2 = _t1(h1, agg1, W1a, b1a.reshape(1, _H), W1b, b1b.reshape(1, _H))

    agg2 = _agg_half(h2, idx_tab, zeros32)
    score = _t2(h2, agg2, W2a, b2a.reshape(1, _H), W2b, b2b.reshape(1, _H),
                Ws1, bs1.reshape(1, _H), Ws2, bs2.reshape(1, 1))
    return score[0, 0]


# speedup vs baseline: 5.4480x; 1.3747x over previous
import functools

import jax
import jax.numpy as jnp
from jax import lax
from jax.experimental import pallas as pl
from jax.experimental.pallas import tpu as pltpu
from jax.experimental.pallas import tpu_sc as plsc

_N = 50000
_NP = 50048
_E = 800000
_H = 64
_HALF = 32

_NSUB = 16
_CH = 128
_EP = 819200
_NCHT = 6400
_NCH = _NCHT // _NSUB
_GRP = 40
_NGRP = _NCH // _GRP
_NBUF = 4
_NPS = _NP // _NSUB
_NP2 = 50176
_NPS2 = _NP2 // _NSUB
_TRASH = 50048


def _make_agg(F):
    mesh = plsc.VectorSubcoreMesh(core_axis_name="c", subcore_axis_name="s")

    @functools.partial(
        pl.kernel,
        mesh=mesh,
        compiler_params=pltpu.CompilerParams(use_tc_tiling_on_sc=False),
        out_type=jax.ShapeDtypeStruct((2, _NP, F), jnp.float32),
        scratch_types=[
            pltpu.VMEM((400,), jnp.int32),
            pltpu.VMEM((_GRP, 2, _CH), jnp.int32),
            pltpu.VMEM((_NBUF, _CH, F), jnp.float32),
            pltpu.VMEM((64, max(F, 16)), jnp.float32),
            pltpu.VMEM_SHARED((_NP2, F), jnp.float32),
            pltpu.SemaphoreType.DMA,
            pltpu.SemaphoreType.DMA,
            pltpu.SemaphoreType.DMA,
            pltpu.SemaphoreType.DMA,
            pltpu.SemaphoreType.DMA,
            pltpu.SemaphoreType.DMA,
            pltpu.SemaphoreType.DMA,
            pltpu.SemaphoreType.DMA,
            pltpu.SemaphoreType.DMA,
        ],
    )
    def k(h_st, idx_tab, out, ids_v, idx_v, rows_v, zb, agg_sh,
          semI, sg0, sg1, sg2, sg3, ss0, ss1, ss2, ss3):
        sg = [sg0, sg1, sg2, sg3]
        ss = [ss0, ss1, ss2, ss3]
        s = lax.axis_index("s")
        c = lax.axis_index("c")
        z16 = jnp.zeros((16,), jnp.float32)
        for r in range(64):
            for q in range(max(F // 16, 1)):
                zb[r, pl.ds(q * 16, 16)] = z16
        slab0 = s * _NPS2
        zbF = zb if F >= 16 else zb.at[:, pl.ds(0, F)]
        for i in range(_NPS2 // 64):
            pltpu.sync_copy(zbF, agg_sh.at[pl.ds(slab0 + i * 64, 64)])

        iota16 = lax.iota(jnp.int32, 16)
        cid0 = s * _NCH
        for i in range(_NCH // 16):
            ids_v[pl.ds(i * 16, 16)] = iota16 + (cid0 + i * 16)
        plsc.subcore_barrier()

        for cc in range(2):
            @pl.when(c == cc)
            def _():
                h_c = h_st.at[cc]

                def group(g, carry):
                    pltpu.async_copy(
                        idx_tab.at[ids_v.at[pl.ds(g * _GRP, _GRP)]],
                        idx_v, semI).wait()
                    ib = idx_v

                    gd = [None] * _NBUF
                    sd = [None] * _NBUF
                    for t in range(2):
                        gd[t] = pltpu.async_copy(
                            h_c.at[ib.at[t, 0]], rows_v.at[t], sg[t])
                    for t in range(_GRP):
                        b = t % _NBUF
                        gd[b].wait()
                        sd[b] = pltpu.async_copy(
                            rows_v.at[b], agg_sh.at[ib.at[t, 1]],
                            ss[b], add=True)
                        w = t + 2
                        if w < _GRP:
                            bw = w % _NBUF
                            if sd[bw] is not None:
                                sd[bw].wait()
                            gd[bw] = pltpu.async_copy(
                                h_c.at[ib.at[w, 0]], rows_v.at[bw], sg[bw])
                    sd[(_GRP - 2) % _NBUF].wait()
                    sd[(_GRP - 1) % _NBUF].wait()
                    return carry

                lax.fori_loop(0, _NGRP, group, 0)

        plsc.subcore_barrier()
        row0 = s * _NPS
        for cc in range(2):
            @pl.when(c == cc)
            def _():
                pltpu.sync_copy(agg_sh.at[pl.ds(row0, _NPS)],
                                out.at[cc].at[pl.ds(row0, _NPS)])

    return k


_R = 2000
_G = _N // _R


def _mlp(z, Wa, ba, Wb, bb):
    z = jnp.maximum(jnp.dot(z, Wa, preferred_element_type=jnp.float32) + ba, 0.0)
    z = jnp.dot(z, Wb, preferred_element_type=jnp.float32) + bb
    return jnp.maximum(z, 0.0)


def _t0_body(feat, agg, Wa, ba, Wb, bb, out):
    z = feat[...] + agg[...]
    h = _mlp(z, Wa[...], ba[...], Wb[...], bb[...])
    out[0] = h[:, :_HALF]
    out[1] = h[:, _HALF:]


def _t1_body(h_st, agg, Wa, ba, Wb, bb, out):
    z = jnp.concatenate([h_st[0] + agg[0], h_st[1] + agg[1]], axis=1)
    h = _mlp(z, Wa[...], ba[...], Wb[...], bb[...])
    out[0] = h[:, :_HALF]
    out[1] = h[:, _HALF:]


def _t2_body(h_st, agg, Wa, ba, Wb, bb, Ws1, bs1, Ws2, bs2, score, acc):
    i = pl.program_id(0)
    z = jnp.concatenate([h_st[0] + agg[0], h_st[1] + agg[1]], axis=1)
    h = _mlp(z, Wa[...], ba[...], Wb[...], bb[...])
    blk = jnp.sum(h, axis=0, keepdims=True)

    @pl.when(i == 0)
    def _():
        acc[...] = blk

    @pl.when(i > 0)
    def _():
        acc[...] = acc[...] + blk

    @pl.when(i == _G - 1)
    def _():
        hg = acc[...] * (1.0 / _N)
        sv = jnp.maximum(
            jnp.dot(hg, Ws1[...], preferred_element_type=jnp.float32) + bs1[...],
            0.0)
        sc = jnp.dot(sv, Ws2[...], preferred_element_type=jnp.float32) + bs2[...]
        score[...] = jax.nn.sigmoid(sc)


def _full_spec(shape):
    return pl.BlockSpec(shape, lambda i: tuple(0 for _ in shape))


_t0 = pl.pallas_call(
    _t0_body,
    grid=(_G,),
    in_specs=[
        pl.BlockSpec((_R, 2), lambda i: (i, 0)),
        pl.BlockSpec((_R, 2), lambda i: (i, 0)),
        _full_spec((2, _H)),
        _full_spec((1, _H)),
        _full_spec((_H, _H)),
        _full_spec((1, _H)),
    ],
    out_specs=pl.BlockSpec((2, _R, _HALF), lambda i: (0, i, 0)),
    out_shape=jax.ShapeDtypeStruct((2, _NP, _HALF), jnp.float32),
)

_t1 = pl.pallas_call(
    _t1_body,
    grid=(_G,),
    in_specs=[
        pl.BlockSpec((2, _R, _HALF), lambda i: (0, i, 0)),
        pl.BlockSpec((2, _R, _HALF), lambda i: (0, i, 0)),
        _full_spec((_H, _H)),
        _full_spec((1, _H)),
        _full_spec((_H, _H)),
        _full_spec((1, _H)),
    ],
    out_specs=pl.BlockSpec((2, _R, _HALF), lambda i: (0, i, 0)),
    out_shape=jax.ShapeDtypeStruct((2, _NP, _HALF), jnp.float32),
)

_t2 = pl.pallas_call(
    _t2_body,
    grid=(_G,),
    in_specs=[
        pl.BlockSpec((2, _R, _HALF), lambda i: (0, i, 0)),
        pl.BlockSpec((2, _R, _HALF), lambda i: (0, i, 0)),
        _full_spec((_H, _H)),
        _full_spec((1, _H)),
        _full_spec((_H, _H)),
        _full_spec((1, _H)),
        _full_spec((_H, _H)),
        _full_spec((1, _H)),
        _full_spec((_H, 1)),
        _full_spec((1, 1)),
    ],
    out_specs=pl.BlockSpec((1, 1), lambda i: (0, 0)),
    out_shape=jax.ShapeDtypeStruct((1, 1), jnp.float32),
    scratch_shapes=[pltpu.VMEM((1, _H), jnp.float32)],
)

_F0 = 8
_agg_half = _make_agg(_HALF)
_agg_f0 = _make_agg(_F0)


def kernel(edge_index, u_flag, v_flag, W0a, b0a, W0b, b0b,
           W1a, b1a, W1b, b1b, W2a, b2a, W2b, b2b, Ws1, bs1, Ws2, bs2):
    u32 = u_flag.astype(jnp.float32)
    v32 = v_flag.astype(jnp.float32)
    flags2 = jnp.stack([u32, v32], axis=1)
    flags_st = jnp.zeros((2, _NP, _F0), jnp.float32).at[:, :_N, 0].set(
        jnp.stack([u32, v32], axis=0))
    pad = _EP - _E
    src_p = jnp.concatenate([edge_index[0], jnp.zeros((pad,), jnp.int32)])
    dst_p = jnp.concatenate(
        [edge_index[1], jnp.full((pad,), _TRASH, jnp.int32)])
    idx_tab = jnp.stack(
        [src_p.reshape(_NCHT, _CH), dst_p.reshape(_NCHT, _CH)], axis=1)
    agg0 = _agg_f0(flags_st, idx_tab)
    agg0_2 = agg0[:, :_N, 0].T
    h1 = _t0(flags2, agg0_2, W0a, b0a.reshape(1, _H), W0b, b0b.reshape(1, _H))

    agg1 = _agg_half(h1, idx_tab)
    h2 = _t1(h1, agg1, W1a, b1a.reshape(1, _H), W1b, b1b.reshape(1, _H))

    agg2 = _agg_half(h2, idx_tab)
    score = _t2(h2, agg2, W2a, b2a.reshape(1, _H), W2b, b2b.reshape(1, _H),
                Ws1, bs1.reshape(1, _H), Ws2, bs2.reshape(1, 1))
    return score[0, 0]


# speedup vs baseline: 6.7042x; 1.2306x over previous
import functools

import jax
import jax.numpy as jnp
from jax import lax
from jax.experimental import pallas as pl
from jax.experimental.pallas import tpu as pltpu
from jax.experimental.pallas import tpu_sc as plsc

_N = 50000
_NP = 50048
_E = 800000
_H = 64
_HALF = 32

_NSUB = 16
_CH = 128
_EP = 819200
_NCHT = 6400
_NCH = _NCHT // _NSUB
_GRP = 40
_NGRP = _NCH // _GRP
_NBUF = 4
_NPS = _NP // _NSUB
_NP2 = 50176
_NPS2 = _NP2 // _NSUB
_TRASH = 50048


def _make_agg(F):
    mesh = plsc.VectorSubcoreMesh(core_axis_name="c", subcore_axis_name="s")

    @functools.partial(
        pl.kernel,
        mesh=mesh,
        compiler_params=pltpu.CompilerParams(use_tc_tiling_on_sc=False),
        out_type=jax.ShapeDtypeStruct((2, _NP, F), jnp.float32),
        scratch_types=[
            pltpu.VMEM((400,), jnp.int32),
            pltpu.VMEM((_GRP, 2, _CH), jnp.int32),
            pltpu.VMEM((_NBUF, _CH, F), jnp.float32),
            pltpu.VMEM((64, max(F, 16)), jnp.float32),
            pltpu.VMEM_SHARED((_NP2, F), jnp.float32),
            pltpu.SemaphoreType.DMA,
            pltpu.SemaphoreType.DMA,
            pltpu.SemaphoreType.DMA,
            pltpu.SemaphoreType.DMA,
            pltpu.SemaphoreType.DMA,
            pltpu.SemaphoreType.DMA,
            pltpu.SemaphoreType.DMA,
            pltpu.SemaphoreType.DMA,
            pltpu.SemaphoreType.DMA,
        ],
    )
    def k(h_st, idx_tab, out, ids_v, idx_v, rows_v, zb, agg_sh,
          semI, sg0, sg1, sg2, sg3, ss0, ss1, ss2, ss3):
        sg = [sg0, sg1, sg2, sg3]
        ss = [ss0, ss1, ss2, ss3]
        s = lax.axis_index("s")
        c = lax.axis_index("c")
        z16 = jnp.zeros((16,), jnp.float32)
        for r in range(64):
            for q in range(max(F // 16, 1)):
                zb[r, pl.ds(q * 16, 16)] = z16
        slab0 = s * _NPS2
        zbF = zb if F >= 16 else zb.at[:, pl.ds(0, F)]
        for i in range(_NPS2 // 64):
            pltpu.sync_copy(zbF, agg_sh.at[pl.ds(slab0 + i * 64, 64)])

        iota16 = lax.iota(jnp.int32, 16)
        cid0 = s * _NCH
        for i in range(_NCH // 16):
            ids_v[pl.ds(i * 16, 16)] = iota16 + (cid0 + i * 16)
        plsc.subcore_barrier()

        for cc in range(2):
            @pl.when(c == cc)
            def _():
                h_c = h_st.at[cc]

                def group(g, carry):
                    pltpu.async_copy(
                        idx_tab.at[ids_v.at[pl.ds(g * _GRP, _GRP)]],
                        idx_v, semI).wait()
                    ib = idx_v

                    gd = [None] * _NBUF
                    sd = [None] * _NBUF
                    for t in range(2):
                        gd[t] = pltpu.async_copy(
                            h_c.at[ib.at[t, 0]], rows_v.at[t], sg[t])
                    for t in range(_GRP):
                        b = t % _NBUF
                        gd[b].wait()
                        sd[b] = pltpu.async_copy(
                            rows_v.at[b], agg_sh.at[ib.at[t, 1]],
                            ss[b], add=True)
                        w = t + 2
                        if w < _GRP:
                            bw = w % _NBUF
                            if sd[bw] is not None:
                                sd[bw].wait()
                            gd[bw] = pltpu.async_copy(
                                h_c.at[ib.at[w, 0]], rows_v.at[bw], sg[bw])
                    sd[(_GRP - 2) % _NBUF].wait()
                    sd[(_GRP - 1) % _NBUF].wait()
                    return carry

                lax.fori_loop(0, _NGRP, group, 0)

        plsc.subcore_barrier()
        row0 = s * _NPS
        for cc in range(2):
            @pl.when(c == cc)
            def _():
                pltpu.sync_copy(agg_sh.at[pl.ds(row0, _NPS)],
                                out.at[cc].at[pl.ds(row0, _NPS)])

    return k


_R = 2176
_G = _NP // _R
_RP = _R // 4
_NPP = _NP // 4


def _unpack_cols(h, c):
    return jnp.concatenate([h[:, 64 * k + 32 * c: 64 * k + 32 * c + 32]
                            for k in range(4)], axis=1)


def _t0_body(feat, agg, Wa, ba, Wb, bb, out):
    z = feat[...] + agg[...]
    z = jnp.maximum(
        jnp.dot(z, Wa[...], preferred_element_type=jnp.float32) + ba[...], 0.0)
    z = jnp.dot(z, Wb[...], preferred_element_type=jnp.float32) + bb[...]
    h = jnp.maximum(z, 0.0)
    out[0] = _unpack_cols(h, 0)
    out[1] = _unpack_cols(h, 1)


def _mlp_packed(x0, x1, Wa0, Wa1, ba, Wb, bb):
    z = jnp.dot(x0, Wa0, preferred_element_type=jnp.float32)
    z = z + jnp.dot(x1, Wa1, preferred_element_type=jnp.float32)
    z = jnp.maximum(z + ba, 0.0)
    z = jnp.dot(z, Wb, preferred_element_type=jnp.float32) + bb
    return jnp.maximum(z, 0.0)


def _t1_body(h_st, agg, Wa0, Wa1, ba, Wb, bb, out):
    h = _mlp_packed(h_st[0] + agg[0], h_st[1] + agg[1],
                    Wa0[...], Wa1[...], ba[...], Wb[...], bb[...])
    out[0] = _unpack_cols(h, 0)
    out[1] = _unpack_cols(h, 1)


def _t2_body(h_st, agg, Wa0, Wa1, ba, Wb, bb, Ws1, bs1, Ws2, bs2,
             score, acc):
    i = pl.program_id(0)
    h = _mlp_packed(h_st[0] + agg[0], h_st[1] + agg[1],
                    Wa0[...], Wa1[...], ba[...], Wb[...], bb[...])

    @pl.when(i == _G - 1)
    def _():
        rid = lax.broadcasted_iota(jnp.int32, (_RP, 1), 0)
        nvalid = _RP - (_NP - _N) // 4
        h_m = jnp.where(rid < nvalid, h, 0.0)
        acc[...] = acc[...] + jnp.sum(h_m, axis=0, keepdims=True)
        a = acc[...]
        hg = (a[:, 0:64] + a[:, 64:128] + a[:, 128:192]
              + a[:, 192:256]) * (1.0 / _N)
        sv = jnp.maximum(
            jnp.dot(hg, Ws1[...], preferred_element_type=jnp.float32)
            + bs1[...], 0.0)
        sc = jnp.dot(sv, Ws2[...], preferred_element_type=jnp.float32) + bs2[...]
        score[...] = jax.nn.sigmoid(sc)

    @pl.when(i == 0)
    def _():
        acc[...] = jnp.sum(h, axis=0, keepdims=True)

    @pl.when((i > 0) & (i < _G - 1))
    def _():
        acc[...] = acc[...] + jnp.sum(h, axis=0, keepdims=True)


def _full_spec(shape):
    return pl.BlockSpec(shape, lambda i: tuple(0 for _ in shape))


_t0 = pl.pallas_call(
    _t0_body,
    grid=(_G,),
    in_specs=[
        pl.BlockSpec((_RP, 8), lambda i: (i, 0)),
        pl.BlockSpec((_RP, 8), lambda i: (i, 0)),
        _full_spec((8, 256)),
        _full_spec((1, 256)),
        _full_spec((256, 256)),
        _full_spec((1, 256)),
    ],
    out_specs=pl.BlockSpec((2, _RP, 128), lambda i: (0, i, 0)),
    out_shape=jax.ShapeDtypeStruct((2, _NPP, 128), jnp.float32),
)

_t1 = pl.pallas_call(
    _t1_body,
    grid=(_G,),
    in_specs=[
        pl.BlockSpec((2, _RP, 128), lambda i: (0, i, 0)),
        pl.BlockSpec((2, _RP, 128), lambda i: (0, i, 0)),
        _full_spec((128, 256)),
        _full_spec((128, 256)),
        _full_spec((1, 256)),
        _full_spec((256, 256)),
        _full_spec((1, 256)),
    ],
    out_specs=pl.BlockSpec((2, _RP, 128), lambda i: (0, i, 0)),
    out_shape=jax.ShapeDtypeStruct((2, _NPP, 128), jnp.float32),
)

_t2 = pl.pallas_call(
    _t2_body,
    grid=(_G,),
    in_specs=[
        pl.BlockSpec((2, _RP, 128), lambda i: (0, i, 0)),
        pl.BlockSpec((2, _RP, 128), lambda i: (0, i, 0)),
        _full_spec((128, 256)),
        _full_spec((128, 256)),
        _full_spec((1, 256)),
        _full_spec((256, 256)),
        _full_spec((1, 256)),
        _full_spec((64, 64)),
        _full_spec((1, 64)),
        _full_spec((64, 1)),
        _full_spec((1, 1)),
    ],
    out_specs=pl.BlockSpec((1, 1), lambda i: (0, 0)),
    out_shape=jax.ShapeDtypeStruct((1, 1), jnp.float32),
    scratch_shapes=[pltpu.VMEM((1, 256), jnp.float32)],
)

_F0 = 8
_agg_half = _make_agg(_HALF)
_agg_f0 = _make_agg(_F0)


def kernel(edge_index, u_flag, v_flag, W0a, b0a, W0b, b0b,
           W1a, b1a, W1b, b1b, W2a, b2a, W2b, b2b, Ws1, bs1, Ws2, bs2):
    u32 = u_flag.astype(jnp.float32)
    v32 = v_flag.astype(jnp.float32)
    eye4 = jnp.eye(4, dtype=jnp.float32)

    def bd4(W):
        return jnp.kron(eye4, W)

    def tile4(b):
        return jnp.tile(b, 4).reshape(1, -1)

    flags2 = jnp.stack([u32, v32], axis=1)
    flags2p = jnp.concatenate(
        [flags2, jnp.zeros((_NP - _N, 2), jnp.float32)]).reshape(_NPP, 8)
    flags_st = jnp.zeros((2, _NP, _F0), jnp.float32).at[:, :_N, 0].set(
        jnp.stack([u32, v32], axis=0))
    pad = _EP - _E
    src_p = jnp.concatenate([edge_index[0], jnp.zeros((pad,), jnp.int32)])
    dst_p = jnp.concatenate(
        [edge_index[1], jnp.full((pad,), _TRASH, jnp.int32)])
    idx_tab = jnp.stack(
        [src_p.reshape(_NCHT, _CH), dst_p.reshape(_NCHT, _CH)], axis=1)

    agg0 = _agg_f0(flags_st, idx_tab)
    agg0_2 = agg0[:, :, 0].T
    agg0p = agg0_2.reshape(_NPP, 8)
    h1 = _t0(flags2p, agg0p,
             bd4(W0a), tile4(b0a), bd4(W0b), tile4(b0b))

    agg1 = _agg_half(h1.reshape(2, _NP, _HALF), idx_tab).reshape(2, _NPP, 128)
    h2 = _t1(h1, agg1, bd4(W1a[:_HALF]), bd4(W1a[_HALF:]), tile4(b1a),
             bd4(W1b), tile4(b1b))

    agg2 = _agg_half(h2.reshape(2, _NP, _HALF), idx_tab).reshape(2, _NPP, 128)
    score = _t2(h2, agg2, bd4(W2a[:_HALF]), bd4(W2a[_HALF:]), tile4(b2a),
                bd4(W2b), tile4(b2b),
                Ws1, bs1.reshape(1, _H), Ws2, bs2.reshape(1, 1))
    return score[0, 0]


# speedup vs baseline: 6.7399x; 1.0053x over previous
import functools

import jax
import jax.numpy as jnp
from jax import lax
from jax.experimental import pallas as pl
from jax.experimental.pallas import tpu as pltpu
from jax.experimental.pallas import tpu_sc as plsc

_N = 50000
_NP = 50048
_E = 800000
_H = 64
_HALF = 32

_NSUB = 16
_CH = 128
_EP = 819200
_NCHT = 6400
_NCH = _NCHT // _NSUB
_GRP = 40
_NGRP = _NCH // _GRP
_NBUF = 4
_NPS = _NP // _NSUB
_NP2 = 50176
_NPS2 = _NP2 // _NSUB
_TRASH = 50048


def _make_agg(F):
    mesh = plsc.VectorSubcoreMesh(core_axis_name="c", subcore_axis_name="s")

    @functools.partial(
        pl.kernel,
        mesh=mesh,
        compiler_params=pltpu.CompilerParams(use_tc_tiling_on_sc=False),
        out_type=jax.ShapeDtypeStruct((2, _NP, F), jnp.float32),
        scratch_types=[
            pltpu.VMEM((_NGRP, 48), jnp.int32),
            pltpu.VMEM((_GRP, 2, _CH), jnp.int32),
            pltpu.VMEM((_NBUF, _CH, F), jnp.float32),
            pltpu.VMEM((64, max(F, 16)), jnp.float32),
            pltpu.VMEM_SHARED((_NP2, F), jnp.float32),
            pltpu.SemaphoreType.DMA,
            pltpu.SemaphoreType.DMA,
            pltpu.SemaphoreType.DMA,
            pltpu.SemaphoreType.DMA,
            pltpu.SemaphoreType.DMA,
            pltpu.SemaphoreType.DMA,
            pltpu.SemaphoreType.DMA,
            pltpu.SemaphoreType.DMA,
            pltpu.SemaphoreType.DMA,
        ],
    )
    def k(h_st, idx_tab, out, ids_v, idx_v, rows_v, zb, agg_sh,
          semI, sg0, sg1, sg2, sg3, ss0, ss1, ss2, ss3):
        sg = [sg0, sg1, sg2, sg3]
        ss = [ss0, ss1, ss2, ss3]
        s = lax.axis_index("s")
        c = lax.axis_index("c")
        z16 = jnp.zeros((16,), jnp.float32)
        for r in range(64):
            for q in range(max(F // 16, 1)):
                zb[r, pl.ds(q * 16, 16)] = z16
        slab0 = s * _NPS2
        zbF = zb if F >= 16 else zb.at[:, pl.ds(0, F)]
        zd = [pltpu.async_copy(zbF, agg_sh.at[pl.ds(slab0 + i * 64, 64)], semI)
              for i in range(_NPS2 // 64)]
        for d in zd:
            d.wait()

        iota16 = lax.iota(jnp.int32, 16)
        cid0 = s * _NCH
        for g in range(_NGRP):
            for hh in range(3):
                ids_v[g, pl.ds(hh * 16, 16)] = iota16 + (
                    cid0 + g * _GRP + hh * 16)
        plsc.subcore_barrier()

        for cc in range(2):
            @pl.when(c == cc)
            def _():
                h_c = h_st.at[cc]

                def group(g, carry):
                    pltpu.async_copy(
                        idx_tab.at[ids_v.at[g, pl.ds(0, _GRP)]],
                        idx_v, semI).wait()
                    ib = idx_v

                    gd = [None] * _NBUF
                    sd = [None] * _NBUF
                    for t in range(2):
                        gd[t] = pltpu.async_copy(
                            h_c.at[ib.at[t, 0]], rows_v.at[t], sg[t])
                    for t in range(_GRP):
                        b = t % _NBUF
                        gd[b].wait()
                        sd[b] = pltpu.async_copy(
                            rows_v.at[b], agg_sh.at[ib.at[t, 1]],
                            ss[b], add=True)
                        w = t + 2
                        if w < _GRP:
                            bw = w % _NBUF
                            if sd[bw] is not None:
                                sd[bw].wait()
                            gd[bw] = pltpu.async_copy(
                                h_c.at[ib.at[w, 0]], rows_v.at[bw], sg[bw])
                    for t in range(_GRP - _NBUF, _GRP):
                        sd[t % _NBUF].wait()
                    return carry

                lax.fori_loop(0, _NGRP, group, 0)

        plsc.subcore_barrier()
        row0 = s * _NPS
        for cc in range(2):
            @pl.when(c == cc)
            def _():
                pltpu.sync_copy(agg_sh.at[pl.ds(row0, _NPS)],
                                out.at[cc].at[pl.ds(row0, _NPS)])

    return k


_R = 2176
_G = _NP // _R
_RP = _R // 4
_NPP = _NP // 4


def _unpack_cols(h, c):
    return jnp.concatenate([h[:, 64 * k + 32 * c: 64 * k + 32 * c + 32]
                            for k in range(4)], axis=1)


def _t0_body(feat, agg, Wa, ba, Wb, bb, out):
    z = feat[...] + agg[...]
    z = jnp.maximum(
        jnp.dot(z, Wa[...], preferred_element_type=jnp.float32) + ba[...], 0.0)
    z = jnp.dot(z, Wb[...], preferred_element_type=jnp.float32) + bb[...]
    h = jnp.maximum(z, 0.0)
    out[0] = _unpack_cols(h, 0)
    out[1] = _unpack_cols(h, 1)


def _mlp_packed(x0, x1, Wa0, Wa1, ba, Wb, bb):
    z = jnp.dot(x0, Wa0, preferred_element_type=jnp.float32)
    z = z + jnp.dot(x1, Wa1, preferred_element_type=jnp.float32)
    z = jnp.maximum(z + ba, 0.0)
    z = jnp.dot(z, Wb, preferred_element_type=jnp.float32) + bb
    return jnp.maximum(z, 0.0)


def _t1_body(h_st, agg, Wa0, Wa1, ba, Wb, bb, out):
    h = _mlp_packed(h_st[0] + agg[0], h_st[1] + agg[1],
                    Wa0[...], Wa1[...], ba[...], Wb[...], bb[...])
    out[0] = _unpack_cols(h, 0)
    out[1] = _unpack_cols(h, 1)


def _t2_body(h_st, agg, Wa0, Wa1, ba, Wb, bb, Ws1, bs1, Ws2, bs2,
             score, acc):
    i = pl.program_id(0)
    h = _mlp_packed(h_st[0] + agg[0], h_st[1] + agg[1],
                    Wa0[...], Wa1[...], ba[...], Wb[...], bb[...])

    @pl.when(i == _G - 1)
    def _():
        rid = lax.broadcasted_iota(jnp.int32, (_RP, 1), 0)
        nvalid = _RP - (_NP - _N) // 4
        h_m = jnp.where(rid < nvalid, h, 0.0)
        acc[...] = acc[...] + jnp.sum(h_m, axis=0, keepdims=True)
        a = acc[...]
        hg = (a[:, 0:64] + a[:, 64:128] + a[:, 128:192]
              + a[:, 192:256]) * (1.0 / _N)
        sv = jnp.maximum(
            jnp.dot(hg, Ws1[...], preferred_element_type=jnp.float32)
            + bs1[...], 0.0)
        sc = jnp.dot(sv, Ws2[...], preferred_element_type=jnp.float32) + bs2[...]
        score[...] = jax.nn.sigmoid(sc)

    @pl.when(i == 0)
    def _():
        acc[...] = jnp.sum(h, axis=0, keepdims=True)

    @pl.when((i > 0) & (i < _G - 1))
    def _():
        acc[...] = acc[...] + jnp.sum(h, axis=0, keepdims=True)


def _full_spec(shape):
    return pl.BlockSpec(shape, lambda i: tuple(0 for _ in shape))


_t0 = pl.pallas_call(
    _t0_body,
    grid=(_G,),
    in_specs=[
        pl.BlockSpec((_RP, 8), lambda i: (i, 0)),
        pl.BlockSpec((_RP, 8), lambda i: (i, 0)),
        _full_spec((8, 256)),
        _full_spec((1, 256)),
        _full_spec((256, 256)),
        _full_spec((1, 256)),
    ],
    out_specs=pl.BlockSpec((2, _RP, 128), lambda i: (0, i, 0)),
    out_shape=jax.ShapeDtypeStruct((2, _NPP, 128), jnp.float32),
)

_t1 = pl.pallas_call(
    _t1_body,
    grid=(_G,),
    in_specs=[
        pl.BlockSpec((2, _RP, 128), lambda i: (0, i, 0)),
        pl.BlockSpec((2, _RP, 128), lambda i: (0, i, 0)),
        _full_spec((128, 256)),
        _full_spec((128, 256)),
        _full_spec((1, 256)),
        _full_spec((256, 256)),
        _full_spec((1, 256)),
    ],
    out_specs=pl.BlockSpec((2, _RP, 128), lambda i: (0, i, 0)),
    out_shape=jax.ShapeDtypeStruct((2, _NPP, 128), jnp.float32),
)

_t2 = pl.pallas_call(
    _t2_body,
    grid=(_G,),
    in_specs=[
        pl.BlockSpec((2, _RP, 128), lambda i: (0, i, 0)),
        pl.BlockSpec((2, _RP, 128), lambda i: (0, i, 0)),
        _full_spec((128, 256)),
        _full_spec((128, 256)),
        _full_spec((1, 256)),
        _full_spec((256, 256)),
        _full_spec((1, 256)),
        _full_spec((64, 64)),
        _full_spec((1, 64)),
        _full_spec((64, 1)),
        _full_spec((1, 1)),
    ],
    out_specs=pl.BlockSpec((1, 1), lambda i: (0, 0)),
    out_shape=jax.ShapeDtypeStruct((1, 1), jnp.float32),
    scratch_shapes=[pltpu.VMEM((1, 256), jnp.float32)],
)

_F0 = 8
_agg_half = _make_agg(_HALF)
_agg_f0 = _make_agg(_F0)


def kernel(edge_index, u_flag, v_flag, W0a, b0a, W0b, b0b,
           W1a, b1a, W1b, b1b, W2a, b2a, W2b, b2b, Ws1, bs1, Ws2, bs2):
    u32 = u_flag.astype(jnp.float32)
    v32 = v_flag.astype(jnp.float32)
    eye4 = jnp.eye(4, dtype=jnp.float32)

    def bd4(W):
        return jnp.kron(eye4, W)

    def tile4(b):
        return jnp.tile(b, 4).reshape(1, -1)

    flags2 = jnp.stack([u32, v32], axis=1)
    flags2p = jnp.concatenate(
        [flags2, jnp.zeros((_NP - _N, 2), jnp.float32)]).reshape(_NPP, 8)
    flags_st = jnp.zeros((2, _NP, _F0), jnp.float32).at[:, :_N, 0].set(
        jnp.stack([u32, v32], axis=0))
    pad = _EP - _E
    src_p = jnp.concatenate([edge_index[0], jnp.zeros((pad,), jnp.int32)])
    dst_p = jnp.concatenate(
        [edge_index[1],
         _TRASH + (jnp.arange(pad, dtype=jnp.int32) % 128)])
    idx_tab = jnp.stack(
        [src_p.reshape(_NCHT, _CH), dst_p.reshape(_NCHT, _CH)], axis=1)

    agg0 = _agg_f0(flags_st, idx_tab)
    agg0_2 = agg0[:, :, 0].T
    agg0p = agg0_2.reshape(_NPP, 8)
    h1 = _t0(flags2p, agg0p,
             bd4(W0a), tile4(b0a), bd4(W0b), tile4(b0b))

    agg1 = _agg_half(h1.reshape(2, _NP, _HALF), idx_tab).reshape(2, _NPP, 128)
    h2 = _t1(h1, agg1, bd4(W1a[:_HALF]), bd4(W1a[_HALF:]), tile4(b1a),
             bd4(W1b), tile4(b1b))

    agg2 = _agg_half(h2.reshape(2, _NP, _HALF), idx_tab).reshape(2, _NPP, 128)
    score = _t2(h2, agg2, bd4(W2a[:_HALF]), bd4(W2a[_HALF:]), tile4(b2a),
                bd4(W2b), tile4(b2b),
                Ws1, bs1.reshape(1, _H), Ws2, bs2.reshape(1, 1))
    return score[0, 0]
